# baseline (speedup 1.0000x reference)
import jax
import jax.numpy as jnp
from jax import lax
from jax.experimental import pallas as pl
from jax.experimental.pallas import tpu as pltpu
from jax.experimental.pallas import tpu_sc as plsc

VOCAB = 1000000
DIM = 64
B = 16384
NNEG = 5

NC = 2
NS = 16
NW = NC * NS
BPW = B // NW
C = 128
NCHUNK = BPW // C
NG = C // 16
DB = 8


def _sc_body(stag_hbm, w_hbm, tgt_hbm, ctx_hbm, negf_hbm,
             pos_out, neg_out, pred_out,
             it0, ic0, in00, in10, in20, in30, in40,
             it1, ic1, in01, in11, in21, in31, in41,
             ur0, vr0, nr00, nr10, nr20, nr30, nr40,
             ur1, vr1, nr01, nr11, nr21, nr31, nr41,
             w_v, pos_v, pred_v, nv0, nv1, nv2, nv3, nv4,
             sem0, sem1):
    idx = [[it0, ic0, in00, in10, in20, in30, in40],
           [it1, ic1, in01, in11, in21, in31, in41]]
    rows = [[ur0, vr0, nr00, nr10, nr20, nr30, nr40],
            [ur1, vr1, nr01, nr11, nr21, nr31, nr41]]
    sems = [sem0, sem1]
    neg_v = [nv0, nv1, nv2, nv3, nv4]

    wid = lax.axis_index("s") * NC + lax.axis_index("c")
    base_w = wid * BPW

    pltpu.sync_copy(w_hbm, w_v)

    iota16 = lax.broadcasted_iota(jnp.int32, (16,), 0)
    zero16 = jnp.full((16,), 0, jnp.int32)

    def load_idx(ci, p):
        base = base_w + ci * C
        pltpu.sync_copy(tgt_hbm.at[pl.ds(base, C)], idx[p][0])
        pltpu.sync_copy(ctx_hbm.at[pl.ds(base, C)], idx[p][1])
        for n in range(NNEG):
            pltpu.sync_copy(negf_hbm.at[pl.ds(n * B + base, C)],
                            idx[p][2 + n])

    def fire_rows(p):
        return [pltpu.async_copy(stag_hbm.at[idx[p][t]], rows[p][t], sems[p])
                for t in range(7)]

    load_idx(0, 0)
    inflight = fire_rows(0)

    for ci in range(NCHUNK):
        p = ci % 2
        if ci + 1 < NCHUNK:
            load_idx(ci + 1, 1 - p)
            nxt = fire_rows(1 - p)
        else:
            nxt = None
        for cp in inflight:
            cp.wait()
        inflight = nxt

        u_rows, v_rows = rows[p][0], rows[p][1]
        n_rows = rows[p][2:]

        def group(g, _):
            ridx = iota16 + g * 16

            def dblock(db, carry):
                acc_pos, acc_pred, a0, a1, a2, a3, a4 = carry
                d0 = db * DB
                for k in range(DB):
                    col = zero16 + (d0 + k)
                    u_d = plsc.load_gather(u_rows, [ridx, col])
                    v_d = plsc.load_gather(v_rows, [ridx, col])
                    w_d = plsc.load_gather(w_v, [col])
                    acc_pos = acc_pos + u_d * v_d
                    acc_pred = acc_pred + u_d * w_d
                    a0 = a0 + u_d * plsc.load_gather(n_rows[0], [ridx, col])
                    a1 = a1 + u_d * plsc.load_gather(n_rows[1], [ridx, col])
                    a2 = a2 + u_d * plsc.load_gather(n_rows[2], [ridx, col])
                    a3 = a3 + u_d * plsc.load_gather(n_rows[3], [ridx, col])
                    a4 = a4 + u_d * plsc.load_gather(n_rows[4], [ridx, col])
                return (acc_pos, acc_pred, a0, a1, a2, a3, a4)

            z = jnp.zeros((16,), jnp.float32)
            acc = lax.fori_loop(0, DIM // DB, dblock, (z, z, z, z, z, z, z))
            sl = pl.ds(pl.multiple_of(g * 16, 16), 16)
            pos_v[sl] = acc[0]
            pred_v[sl] = acc[1]
            for n in range(NNEG):
                neg_v[n][sl] = acc[2 + n]
            return 0

        lax.fori_loop(0, NG, group, 0)

        base = base_w + ci * C
        pltpu.sync_copy(pos_v, pos_out.at[pl.ds(base, C)])
        pltpu.sync_copy(pred_v, pred_out.at[pl.ds(base, C)])
        for n in range(NNEG):
            pltpu.sync_copy(neg_v[n], neg_out.at[pl.ds(n * B + base, C)])


@jax.jit
def _sc_scores(stag, w_flat, tgt, ctx, negf):
    mesh = plsc.VectorSubcoreMesh(core_axis_name="c", subcore_axis_name="s")
    f = pl.kernel(
        _sc_body,
        out_type=(
            jax.ShapeDtypeStruct((B,), jnp.float32),
            jax.ShapeDtypeStruct((NNEG * B,), jnp.float32),
            jax.ShapeDtypeStruct((B,), jnp.float32),
        ),
        mesh=mesh,
        scratch_types=(
            [pltpu.VMEM((C,), jnp.int32)] * 14
            + [pltpu.VMEM((C, DIM), jnp.float32)] * 14
            + [pltpu.VMEM((DIM,), jnp.float32)]
            + [pltpu.VMEM((C,), jnp.float32)] * 7
            + [pltpu.SemaphoreType.DMA, pltpu.SemaphoreType.DMA]
        ),
        compiler_params=pltpu.CompilerParams(
            needs_layout_passes=False, use_tc_tiling_on_sc=False),
    )
    return f(stag, w_flat, tgt, ctx, negf)


def _tc_body(pos_ref, neg_ref, pred_ref, b_ref, loss_ref, pred_out_ref):
    pos = jnp.clip(pos_ref[...], -10.0, 10.0)
    neg = jnp.clip(neg_ref[...], -10.0, 10.0)
    loss_pos = jnp.log1p(jnp.exp(-pos))
    loss_neg = jnp.log1p(jnp.exp(neg))
    total = jnp.sum(loss_pos) + jnp.sum(loss_neg)
    loss_ref[...] = jnp.reshape(total / B, (1, 1))
    pred_out_ref[...] = pred_ref[...] + b_ref[...]


@jax.jit
def _tc_finalize(pos, neg, pred, b):
    loss, pred_out = pl.pallas_call(
        _tc_body,
        out_shape=(
            jax.ShapeDtypeStruct((1, 1), jnp.float32),
            jax.ShapeDtypeStruct((B // 128, 128), jnp.float32),
        ),
    )(pos.reshape(B // 128, 128), neg.reshape(NNEG * (B // 128), 128),
      pred.reshape(B // 128, 128), b.reshape(1, 1))
    return loss[0, 0], pred_out.reshape(B)


def kernel(u_weight, v_weight, W, b, target_word, context_words, neg_words):
    tgt = target_word.astype(jnp.int32)
    ctx = context_words.astype(jnp.int32) + VOCAB
    negf = neg_words.astype(jnp.int32).T.reshape(NNEG * B) + VOCAB
    stag = jnp.concatenate([u_weight, v_weight], axis=0)
    w_flat = W.reshape(DIM).astype(jnp.float32)
    pos, neg, pred = _sc_scores(stag, w_flat, tgt, ctx, negf)
    return _tc_finalize(pos, neg, pred, b.astype(jnp.float32))


# speedup vs baseline: 1.8707x; 1.8707x over previous
import jax
import jax.numpy as jnp
from jax import lax
from jax.experimental import pallas as pl
from jax.experimental.pallas import tpu as pltpu
from jax.experimental.pallas import tpu_sc as plsc

VOCAB = 1000000
DIM = 64
B = 16384
NNEG = 5

NC = 2
NS = 16
NW = NC * NS
NBLK = VOCAB // 128
TAIL = NBLK * 128

BPW = B // NW
C = 64
NCHUNK = BPW // C
NG = C // 16
DB = 8

_CP = pltpu.CompilerParams(needs_layout_passes=False, use_tc_tiling_on_sc=True)


def _cv_body(uT, vT, tails, stag,
             bu0, bv0, ov0, bu1, bv1, ov1, tl_v, sem0, sem1, wsem0, wsem1):
    wid = lax.axis_index("s") * NC + lax.axis_index("c")
    iota16 = lax.broadcasted_iota(jnp.int32, (16,), 0)
    z16 = jnp.full((16,), 0, jnp.int32)
    nb = (NBLK - wid + NW - 1) // NW
    npair = (nb + 1) // 2

    def fire(vb, bu, bv, sem):
        for fg in range(8):
            sl = pl.ds(fg * 8, 8)
            pltpu.async_copy(uT.at[sl, pl.ds(vb * 128, 128)], bu.at[sl, :],
                             sem)
            pltpu.async_copy(vT.at[sl, pl.ds(vb * 128, 128)], bv.at[sl, :],
                             sem)

    def drain(bu, bv, sem):
        pltpu.make_async_copy(uT.at[:, pl.ds(0, 128)], bu, sem).wait()
        pltpu.make_async_copy(vT.at[:, pl.ds(0, 128)], bv, sem).wait()

    def transpose_block(bu, bv, ov):
        @plsc.parallel_loop(0, 128, step=1, unroll=8)
        def trans(j2):
            col = z16 + j2
            for k in range(4):
                ridx = iota16 + 16 * k
                uvals = plsc.load_gather(bu, [ridx, col])
                vvals = plsc.load_gather(bv, [ridx, col])
                ov[j2, pl.ds(16 * k, 16)] = uvals
                ov[j2, pl.ds(64 + 16 * k, 16)] = vvals

    def wdrain(ov, wsem):
        pltpu.make_async_copy(ov, stag.at[pl.ds(0, 128), :], wsem).wait()

    fire(wid, bu0, bv0, sem0)

    def pair(t, _):
        j0 = 2 * t
        j1 = j0 + 1
        vb0 = wid + j0 * NW

        @pl.when(j1 < nb)
        def _():
            fire(vb0 + NW, bu1, bv1, sem1)

        drain(bu0, bv0, sem0)

        @pl.when(t > 0)
        def _():
            wdrain(ov0, wsem0)

        transpose_block(bu0, bv0, ov0)
        pltpu.async_copy(ov0, stag.at[pl.ds(vb0 * 128, 128), :], wsem0)

        @pl.when(j0 + 2 < nb)
        def _():
            fire(vb0 + 2 * NW, bu0, bv0, sem0)

        @pl.when(j1 < nb)
        def _():
            drain(bu1, bv1, sem1)

            @pl.when(t > 0)
            def _():
                wdrain(ov1, wsem1)

            transpose_block(bu1, bv1, ov1)
            pltpu.async_copy(ov1, stag.at[pl.ds((vb0 + NW) * 128, 128), :],
                             wsem1)

        return 0

    lax.fori_loop(0, npair, pair, 0)
    wdrain(ov0, wsem0)

    @pl.when(nb > 1)
    def _():
        wdrain(ov1, wsem1)

    @pl.when(wid == 0)
    def _():
        pltpu.sync_copy(tails, tl_v)
        pltpu.sync_copy(tl_v, stag.at[pl.ds(TAIL, VOCAB - TAIL), :])


def _sc_body(stag_hbm, w_hbm, tgt_hbm, ctx_hbm, negf_hbm,
             pos_out, neg_out, pred_out,
             it0, ic0, in00, in10, in20, in30, in40,
             it1, ic1, in01, in11, in21, in31, in41,
             ur0, vr0, nr00, nr10, nr20, nr30, nr40,
             ur1, vr1, nr01, nr11, nr21, nr31, nr41,
             w_v, pos_v, pred_v, nv0, nv1, nv2, nv3, nv4,
             sem0, sem1):
    idx = [[it0, ic0, in00, in10, in20, in30, in40],
           [it1, ic1, in01, in11, in21, in31, in41]]
    rows = [[ur0, vr0, nr00, nr10, nr20, nr30, nr40],
            [ur1, vr1, nr01, nr11, nr21, nr31, nr41]]
    sems = [sem0, sem1]
    neg_v = [nv0, nv1, nv2, nv3, nv4]

    wid = lax.axis_index("s") * NC + lax.axis_index("c")
    base_w = wid * BPW

    pltpu.sync_copy(w_hbm, w_v)

    iota16 = lax.broadcasted_iota(jnp.int32, (16,), 0)
    zero16 = jnp.full((16,), 0, jnp.int32)

    def load_idx(ci, p):
        base = base_w + ci * C
        pltpu.sync_copy(tgt_hbm.at[pl.ds(base, C)], idx[p][0])
        pltpu.sync_copy(ctx_hbm.at[pl.ds(base, C)], idx[p][1])
        for n in range(NNEG):
            pltpu.sync_copy(negf_hbm.at[pl.ds(n * B + base, C)],
                            idx[p][2 + n])

    def fire_rows(p):
        return [pltpu.async_copy(stag_hbm.at[idx[p][t]], rows[p][t], sems[p])
                for t in range(7)]

    load_idx(0, 0)
    inflight = fire_rows(0)

    for ci in range(NCHUNK):
        p = ci % 2
        if ci + 1 < NCHUNK:
            load_idx(ci + 1, 1 - p)
            nxt = fire_rows(1 - p)
        else:
            nxt = None
        for cp in inflight:
            cp.wait()
        inflight = nxt

        u_rows = rows[p][0]
        v_rows = rows[p][1]
        n_rows = rows[p][2:]

        def group(g, _):
            ridx = iota16 + g * 16

            def dblock(db, carry):
                acc_pos, acc_pred, a0, a1, a2, a3, a4 = carry
                d0 = db * DB
                for k in range(DB):
                    col = zero16 + (d0 + k)
                    col64 = col + 64
                    u_d = plsc.load_gather(u_rows, [ridx, col])
                    v_d = plsc.load_gather(v_rows, [ridx, col64])
                    w_d = plsc.load_gather(w_v, [col])
                    acc_pos = acc_pos + u_d * v_d
                    acc_pred = acc_pred + u_d * w_d
                    a0 = a0 + u_d * plsc.load_gather(n_rows[0], [ridx, col64])
                    a1 = a1 + u_d * plsc.load_gather(n_rows[1], [ridx, col64])
                    a2 = a2 + u_d * plsc.load_gather(n_rows[2], [ridx, col64])
                    a3 = a3 + u_d * plsc.load_gather(n_rows[3], [ridx, col64])
                    a4 = a4 + u_d * plsc.load_gather(n_rows[4], [ridx, col64])
                return (acc_pos, acc_pred, a0, a1, a2, a3, a4)

            z = jnp.zeros((16,), jnp.float32)
            acc = lax.fori_loop(0, DIM // DB, dblock, (z, z, z, z, z, z, z))
            sl = pl.ds(pl.multiple_of(g * 16, 16), 16)
            pos_v[sl] = acc[0]
            pred_v[sl] = acc[1]
            for n in range(NNEG):
                neg_v[n][sl] = acc[2 + n]
            return 0

        lax.fori_loop(0, NG, group, 0)

        base = base_w + ci * C
        pltpu.sync_copy(pos_v, pos_out.at[pl.ds(base, C)])
        pltpu.sync_copy(pred_v, pred_out.at[pl.ds(base, C)])
        for n in range(NNEG):
            pltpu.sync_copy(neg_v[n], neg_out.at[pl.ds(n * B + base, C)])


def _tc_body(pos_ref, neg_ref, pred_ref, b_ref, loss_ref, pred_out_ref):
    pos = jnp.clip(pos_ref[...], -10.0, 10.0)
    neg = jnp.clip(neg_ref[...], -10.0, 10.0)
    loss_pos = jnp.log1p(jnp.exp(-pos))
    loss_neg = jnp.log1p(jnp.exp(neg))
    total = jnp.sum(loss_pos) + jnp.sum(loss_neg)
    loss_ref[...] = jnp.reshape(total / B, (1, 1))
    pred_out_ref[...] = pred_ref[...] + b_ref[...]


@jax.jit
def _pipeline(u_weight, v_weight, W, b, tgt, ctx, negf):
    mesh = plsc.VectorSubcoreMesh(core_axis_name="c", subcore_axis_name="s")

    scores = pl.kernel(
        _sc_body,
        out_type=(
            jax.ShapeDtypeStruct((B,), jnp.float32),
            jax.ShapeDtypeStruct((NNEG * B,), jnp.float32),
            jax.ShapeDtypeStruct((B,), jnp.float32),
        ),
        mesh=mesh,
        scratch_types=(
            [pltpu.VMEM((C,), jnp.int32)] * 14
            + [pltpu.VMEM((C, 128), jnp.float32)] * 14
            + [pltpu.VMEM((DIM,), jnp.float32)]
            + [pltpu.VMEM((C,), jnp.float32)] * 7
            + [pltpu.SemaphoreType.DMA, pltpu.SemaphoreType.DMA]
        ),
        compiler_params=_CP,
    )

    stag = jnp.concatenate([u_weight.T, v_weight.T], axis=0).T
    w_flat = W.reshape(DIM).astype(jnp.float32)
    pos, neg, pred = scores(stag, w_flat, tgt, ctx, negf)

    loss, pred_out = pl.pallas_call(
        _tc_body,
        out_shape=(
            jax.ShapeDtypeStruct((1, 1), jnp.float32),
            jax.ShapeDtypeStruct((B // 128, 128), jnp.float32),
        ),
    )(pos.reshape(B // 128, 128), neg.reshape(NNEG * (B // 128), 128),
      pred.reshape(B // 128, 128),
      b.astype(jnp.float32).reshape(1, 1))
    return loss[0, 0], pred_out.reshape(B)


def kernel(u_weight, v_weight, W, b, target_word, context_words, neg_words):
    tgt = target_word.astype(jnp.int32)
    ctx = context_words.astype(jnp.int32)
    negf = neg_words.astype(jnp.int32).T.reshape(NNEG * B)
    return _pipeline(u_weight, v_weight, W, b, tgt, ctx, negf)


# speedup vs baseline: 1.8730x; 1.0012x over previous
import jax
import jax.numpy as jnp
from jax import lax
from jax.experimental import pallas as pl
from jax.experimental.pallas import tpu as pltpu
from jax.experimental.pallas import tpu_sc as plsc

VOCAB = 1000000
DIM = 64
B = 16384
NNEG = 5

NC = 2
NS = 16
NW = NC * NS
NBLK = VOCAB // 128
TAIL = NBLK * 128

BPW = B // NW
C = 64
NCHUNK = BPW // C
NG = C // 16
DB = 8

_CP = pltpu.CompilerParams(needs_layout_passes=False, use_tc_tiling_on_sc=True)


def _sc_body(stag_hbm, w_hbm, tgt_hbm, ctx_hbm, negf_hbm,
             pos_out, neg_out, pred_out,
             it0, ic0, in00, in10, in20, in30, in40,
             it1, ic1, in01, in11, in21, in31, in41,
             ur0, vr0, nr00, nr10, nr20, nr30, nr40,
             ur1, vr1, nr01, nr11, nr21, nr31, nr41,
             w_v, pos_v, pred_v, nv0, nv1, nv2, nv3, nv4,
             sem0, sem1):
    idx = [[it0, ic0, in00, in10, in20, in30, in40],
           [it1, ic1, in01, in11, in21, in31, in41]]
    rows = [[ur0, vr0, nr00, nr10, nr20, nr30, nr40],
            [ur1, vr1, nr01, nr11, nr21, nr31, nr41]]
    sems = [sem0, sem1]
    neg_v = [nv0, nv1, nv2, nv3, nv4]

    wid = lax.axis_index("s") * NC + lax.axis_index("c")
    base_w = wid * BPW

    pltpu.sync_copy(w_hbm, w_v)

    iota16 = lax.broadcasted_iota(jnp.int32, (16,), 0)
    zero16 = jnp.full((16,), 0, jnp.int32)

    def load_idx(ci, p):
        base = base_w + ci * C
        pltpu.sync_copy(tgt_hbm.at[pl.ds(base, C)], idx[p][0])
        pltpu.sync_copy(ctx_hbm.at[pl.ds(base, C)], idx[p][1])
        for n in range(NNEG):
            pltpu.sync_copy(negf_hbm.at[pl.ds(n * B + base, C)],
                            idx[p][2 + n])

    def fire_rows(p):
        return [pltpu.async_copy(stag_hbm.at[idx[p][t]], rows[p][t], sems[p])
                for t in range(7)]

    load_idx(0, 0)
    inflight = fire_rows(0)

    for ci in range(NCHUNK):
        p = ci % 2
        if ci + 1 < NCHUNK:
            load_idx(ci + 1, 1 - p)
            nxt = fire_rows(1 - p)
        else:
            nxt = None
        for cp in inflight:
            cp.wait()
        inflight = nxt

        u_rows = rows[p][0]
        v_rows = rows[p][1]
        n_rows = rows[p][2:]

        def group(g, _):
            ridx = iota16 + g * 16

            def dblock(db, carry):
                acc_pos, acc_pred, a0, a1, a2, a3, a4 = carry
                d0 = db * DB
                for k in range(DB):
                    col = zero16 + (d0 + k)
                    col64 = col + 64
                    u_d = plsc.load_gather(u_rows, [ridx, col])
                    v_d = plsc.load_gather(v_rows, [ridx, col64])
                    w_d = plsc.load_gather(w_v, [col])
                    acc_pos = acc_pos + u_d * v_d
                    acc_pred = acc_pred + u_d * w_d
                    a0 = a0 + u_d * plsc.load_gather(n_rows[0], [ridx, col64])
                    a1 = a1 + u_d * plsc.load_gather(n_rows[1], [ridx, col64])
                    a2 = a2 + u_d * plsc.load_gather(n_rows[2], [ridx, col64])
                    a3 = a3 + u_d * plsc.load_gather(n_rows[3], [ridx, col64])
                    a4 = a4 + u_d * plsc.load_gather(n_rows[4], [ridx, col64])
                return (acc_pos, acc_pred, a0, a1, a2, a3, a4)

            z = jnp.zeros((16,), jnp.float32)
            acc = lax.fori_loop(0, DIM // DB, dblock, (z, z, z, z, z, z, z))
            sl = pl.ds(pl.multiple_of(g * 16, 16), 16)
            pos_v[sl] = acc[0]
            pred_v[sl] = acc[1]
            for n in range(NNEG):
                neg_v[n][sl] = acc[2 + n]
            return 0

        lax.fori_loop(0, NG, group, 0)

        base = base_w + ci * C
        pltpu.sync_copy(pos_v, pos_out.at[pl.ds(base, C)])
        pltpu.sync_copy(pred_v, pred_out.at[pl.ds(base, C)])
        for n in range(NNEG):
            pltpu.sync_copy(neg_v[n], neg_out.at[pl.ds(n * B + base, C)])


def _tc_body(pos_ref, neg_ref, pred_ref, b_ref, loss_ref, pred_out_ref):
    pos = jnp.clip(pos_ref[...], -10.0, 10.0)
    neg = jnp.clip(neg_ref[...], -10.0, 10.0)
    loss_pos = jnp.log1p(jnp.exp(-pos))
    loss_neg = jnp.log1p(jnp.exp(neg))
    total = jnp.sum(loss_pos) + jnp.sum(loss_neg)
    loss_ref[...] = jnp.reshape(total / B, (1, 1))
    pred_out_ref[...] = pred_ref[...] + b_ref[...]


@jax.jit
def _pipeline(u_weight, v_weight, W, b, tgt, ctx, negf):
    mesh = plsc.VectorSubcoreMesh(core_axis_name="c", subcore_axis_name="s")

    scores = pl.kernel(
        _sc_body,
        out_type=(
            jax.ShapeDtypeStruct((B,), jnp.float32),
            jax.ShapeDtypeStruct((NNEG * B,), jnp.float32),
            jax.ShapeDtypeStruct((B,), jnp.float32),
        ),
        mesh=mesh,
        scratch_types=(
            [pltpu.VMEM((C,), jnp.int32)] * 14
            + [pltpu.VMEM((C, 128), jnp.float32)] * 14
            + [pltpu.VMEM((DIM,), jnp.float32)]
            + [pltpu.VMEM((C,), jnp.float32)] * 7
            + [pltpu.SemaphoreType.DMA, pltpu.SemaphoreType.DMA]
        ),
        compiler_params=_CP,
    )

    stag = jnp.concatenate([u_weight, v_weight], axis=1)
    w_flat = W.reshape(DIM).astype(jnp.float32)
    pos, neg, pred = scores(stag, w_flat, tgt, ctx, negf)

    loss, pred_out = pl.pallas_call(
        _tc_body,
        out_shape=(
            jax.ShapeDtypeStruct((1, 1), jnp.float32),
            jax.ShapeDtypeStruct((B // 128, 128), jnp.float32),
        ),
    )(pos.reshape(B // 128, 128), neg.reshape(NNEG * (B // 128), 128),
      pred.reshape(B // 128, 128),
      b.astype(jnp.float32).reshape(1, 1))
    return loss[0, 0], pred_out.reshape(B)


def kernel(u_weight, v_weight, W, b, target_word, context_words, neg_words):
    tgt = target_word.astype(jnp.int32)
    ctx = context_words.astype(jnp.int32)
    negf = neg_words.astype(jnp.int32).T.reshape(NNEG * B)
    return _pipeline(u_weight, v_weight, W, b, tgt, ctx, negf)
